# baseline (speedup 1.0000x reference)
import functools

import jax
import jax.numpy as jnp
from jax import lax
from jax.experimental import pallas as pl
from jax.experimental.pallas import tpu as pltpu
from jax.experimental.pallas import tpu_sc as plsc

_B = 2048
_H = 64
_W = 64
_NBT = _B // 128
_NWT = _W // 8
_NQ = 4
_IN_ROWS = _H * 3 * _NWT * _NBT * 8
_OUT_ROWS = 6 * _H * _NWT * _NBT * 8
_ITEMS = _H * _NWT * _NQ
_NC, _NS = 2, 16
_NW = _NC * _NS
_IPW = _ITEMS // _NW
_L = 16
_UNROLL = 8
_CHUNK_R = 32
_VECS = _CHUNK_R * 128 // _L


def _embed_body(img, tab, out, tab_v, ia0, ic0, ia1, ic1, ob0, ob1,
                si0, si1, so0, so1):
    wid = lax.axis_index("s") * _NC + lax.axis_index("c")
    t0 = wid * _IPW
    pltpu.sync_copy(tab, tab_v)

    in_bufs = ((ia0, ic0), (ia1, ic1))
    out_bufs = (ob0, ob1)
    in_sems = (si0, si1)
    out_sems = (so0, so1)

    def in_rows(t, c):
        h = t >> 5
        wt = (t >> 2) & 7
        btq = t & 3
        return (((h * 3 + c) * _NWT + wt) * _NBT + btq * _NQ) * 8

    def out_rows(t, c6):
        h = t >> 5
        wt = (t >> 2) & 7
        btq = t & 3
        return (((c6 * _H + h) * _NWT + wt) * _NBT + btq * _NQ) * 8

    def start_in(t, slot):
        ib0, ib2 = in_bufs[slot]
        sem = in_sems[slot]
        pltpu.async_copy(img.at[pl.ds(in_rows(t, 0), _CHUNK_R)], ib0, sem)
        pltpu.async_copy(img.at[pl.ds(in_rows(t, 2), _CHUNK_R)], ib2, sem)

    def wait_in(t, slot):
        ib0, ib2 = in_bufs[slot]
        sem = in_sems[slot]
        pltpu.make_async_copy(img.at[pl.ds(in_rows(t, 0), _CHUNK_R)], ib0,
                              sem).wait()
        pltpu.make_async_copy(img.at[pl.ds(in_rows(t, 2), _CHUNK_R)], ib2,
                              sem).wait()

    def start_out(t, slot):
        ob = out_bufs[slot]
        sem = out_sems[slot]
        for c6 in range(6):
            pltpu.async_copy(
                ob.at[c6], out.at[pl.ds(out_rows(t, c6), _CHUNK_R)], sem)

    def wait_out(t, slot):
        ob = out_bufs[slot]
        sem = out_sems[slot]
        for c6 in range(6):
            pltpu.make_async_copy(
                ob.at[c6], out.at[pl.ds(out_rows(t, c6), _CHUNK_R)],
                sem).wait()

    start_in(t0, 0)
    start_in(t0 + 1, 1)

    tr = [tab_v[c] for c in range(6)]

    def one_item(g, slot):
        t = t0 + g
        ib0, ib2 = in_bufs[slot]
        ob = out_bufs[slot]

        wait_in(t, slot)

        @pl.when(g >= 2)
        def _():
            wait_out(t - 2, slot)

        @plsc.parallel_loop(0, _VECS, 1, unroll=_UNROLL)
        def _(v):
            row = v >> 3
            col = (v & 7) * _L
            obj = ib0[row, pl.ds(col, _L)]
            st = ib2[row, pl.ds(col, _L)]
            for c in range(4):
                ob[c, row, pl.ds(col, _L)] = jnp.take_along_axis(
                    tr[c], obj, axis=0, mode="promise_in_bounds")
            for c in range(4, 6):
                ob[c, row, pl.ds(col, _L)] = jnp.take_along_axis(
                    tr[c], st, axis=0, mode="promise_in_bounds")

        start_out(t, slot)

        @pl.when(g + 2 < _IPW)
        def _():
            start_in(t + 2, slot)

    def pair(p, carry):
        one_item(2 * p, 0)
        one_item(2 * p + 1, 1)
        return carry

    lax.fori_loop(0, _IPW // 2, pair, 0)

    wait_out(t0 + _IPW - 2, 0)
    wait_out(t0 + _IPW - 1, 1)


_sc_embed = functools.partial(
    pl.kernel,
    out_type=jax.ShapeDtypeStruct((_OUT_ROWS, 128), jnp.float32),
    mesh=plsc.VectorSubcoreMesh(core_axis_name="c", subcore_axis_name="s"),
    compiler_params=pltpu.CompilerParams(needs_layout_passes=False),
    scratch_types=[
        pltpu.VMEM((6, _L), jnp.float32),
        pltpu.VMEM((_CHUNK_R, 128), jnp.int32),
        pltpu.VMEM((_CHUNK_R, 128), jnp.int32),
        pltpu.VMEM((_CHUNK_R, 128), jnp.int32),
        pltpu.VMEM((_CHUNK_R, 128), jnp.int32),
        pltpu.VMEM((6, _CHUNK_R, 128), jnp.float32),
        pltpu.VMEM((6, _CHUNK_R, 128), jnp.float32),
        pltpu.SemaphoreType.DMA,
        pltpu.SemaphoreType.DMA,
        pltpu.SemaphoreType.DMA,
        pltpu.SemaphoreType.DMA,
    ],
)(_embed_body)


def kernel(image, object_emb, state_emb):
    x = image.astype(jnp.int32)
    x = x.transpose(1, 3, 2, 0)
    x = x.reshape(_H, 3, _NWT, 8, _NBT, 128)
    x = x.transpose(0, 1, 2, 4, 3, 5)
    imgv = x.reshape(_IN_ROWS, 128)

    tab = jnp.zeros((6, _L), jnp.float32)
    tab = tab.at[0:4, 0:10].set(object_emb[0:10, :].T)
    tab = tab.at[4:6, 0:10].set(state_emb[0:10, :].T)

    outv = _sc_embed(imgv, tab)

    y = outv.reshape(6, _H, _NWT, _NBT, 8, 128)
    y = y.transpose(0, 1, 2, 4, 3, 5)
    y = y.reshape(6, _H, _W, _B)
    return y.transpose(3, 0, 1, 2)


# speedup vs baseline: 1.0030x; 1.0030x over previous
import functools

import jax
import jax.numpy as jnp
from jax import lax
from jax.experimental import pallas as pl
from jax.experimental.pallas import tpu as pltpu
from jax.experimental.pallas import tpu_sc as plsc

_B = 2048
_H = 64
_W = 64
_NBT = _B // 128
_NWT = _W // 8
_NQ = 4
_IN_ROWS = _H * 3 * _NWT * _NBT * 8
_OUT_ROWS = 6 * _H * _NWT * _NBT * 8
_ITEMS = _H * _NWT * _NQ
_NC, _NS = 2, 16
_NW = _NC * _NS
_IPW = _ITEMS // _NW
_L = 16
_UNROLL = 4
_CHUNK_R = 32
_VECS = _CHUNK_R * 128 // _L


def _embed_body(img, tab, out, tab_v, ia0, ic0, ia1, ic1, ob0, ob1,
                si0, si1, so0, so1):
    wid = lax.axis_index("s") * _NC + lax.axis_index("c")
    t0 = wid * _IPW
    pltpu.sync_copy(tab, tab_v)

    in_bufs = ((ia0, ic0), (ia1, ic1))
    out_bufs = (ob0, ob1)
    in_sems = (si0, si1)
    out_sems = (so0, so1)

    def in_rows(t, c):
        h = t >> 5
        wt = (t >> 2) & 7
        btq = t & 3
        return (((h * 3 + c) * _NWT + wt) * _NBT + btq * _NQ) * 8

    def out_rows(t, c6):
        h = t >> 5
        wt = (t >> 2) & 7
        btq = t & 3
        return (((c6 * _H + h) * _NWT + wt) * _NBT + btq * _NQ) * 8

    def start_in(t, slot):
        ib0, ib2 = in_bufs[slot]
        sem = in_sems[slot]
        pltpu.async_copy(img.at[pl.ds(in_rows(t, 0), _CHUNK_R)], ib0, sem)
        pltpu.async_copy(img.at[pl.ds(in_rows(t, 2), _CHUNK_R)], ib2, sem)

    def wait_in(t, slot):
        ib0, ib2 = in_bufs[slot]
        sem = in_sems[slot]
        pltpu.make_async_copy(img.at[pl.ds(in_rows(t, 0), _CHUNK_R)], ib0,
                              sem).wait()
        pltpu.make_async_copy(img.at[pl.ds(in_rows(t, 2), _CHUNK_R)], ib2,
                              sem).wait()

    def start_out(t, slot):
        ob = out_bufs[slot]
        sem = out_sems[slot]
        for c6 in range(6):
            pltpu.async_copy(
                ob.at[c6], out.at[pl.ds(out_rows(t, c6), _CHUNK_R)], sem)

    def wait_out(t, slot):
        ob = out_bufs[slot]
        sem = out_sems[slot]
        for c6 in range(6):
            pltpu.make_async_copy(
                ob.at[c6], out.at[pl.ds(out_rows(t, c6), _CHUNK_R)],
                sem).wait()

    start_in(t0, 0)
    start_in(t0 + 1, 1)

    tr = [tab_v[c] for c in range(6)]

    def one_item(g, slot):
        t = t0 + g
        ib0, ib2 = in_bufs[slot]
        ob = out_bufs[slot]

        wait_in(t, slot)

        @pl.when(g >= 2)
        def _():
            wait_out(t - 2, slot)

        @plsc.parallel_loop(0, _VECS, 1, unroll=_UNROLL)
        def _(v):
            row = v >> 3
            col = (v & 7) * _L
            obj = ib0[row, pl.ds(col, _L)]
            st = ib2[row, pl.ds(col, _L)]
            for c in range(4):
                ob[c, row, pl.ds(col, _L)] = jnp.take_along_axis(
                    tr[c], obj, axis=0, mode="promise_in_bounds")
            for c in range(4, 6):
                ob[c, row, pl.ds(col, _L)] = jnp.take_along_axis(
                    tr[c], st, axis=0, mode="promise_in_bounds")

        start_out(t, slot)

        @pl.when(g + 2 < _IPW)
        def _():
            start_in(t + 2, slot)

    def pair(p, carry):
        one_item(2 * p, 0)
        one_item(2 * p + 1, 1)
        return carry

    lax.fori_loop(0, _IPW // 2, pair, 0)

    wait_out(t0 + _IPW - 2, 0)
    wait_out(t0 + _IPW - 1, 1)


_sc_embed = functools.partial(
    pl.kernel,
    out_type=jax.ShapeDtypeStruct((_OUT_ROWS, 128), jnp.float32),
    mesh=plsc.VectorSubcoreMesh(core_axis_name="c", subcore_axis_name="s"),
    compiler_params=pltpu.CompilerParams(needs_layout_passes=False),
    scratch_types=[
        pltpu.VMEM((6, _L), jnp.float32),
        pltpu.VMEM((_CHUNK_R, 128), jnp.int32),
        pltpu.VMEM((_CHUNK_R, 128), jnp.int32),
        pltpu.VMEM((_CHUNK_R, 128), jnp.int32),
        pltpu.VMEM((_CHUNK_R, 128), jnp.int32),
        pltpu.VMEM((6, _CHUNK_R, 128), jnp.float32),
        pltpu.VMEM((6, _CHUNK_R, 128), jnp.float32),
        pltpu.SemaphoreType.DMA,
        pltpu.SemaphoreType.DMA,
        pltpu.SemaphoreType.DMA,
        pltpu.SemaphoreType.DMA,
    ],
)(_embed_body)


def kernel(image, object_emb, state_emb):
    x = image.astype(jnp.int32)
    x = x.transpose(1, 3, 2, 0)
    x = x.reshape(_H, 3, _NWT, 8, _NBT, 128)
    x = x.transpose(0, 1, 2, 4, 3, 5)
    imgv = x.reshape(_IN_ROWS, 128)

    tab = jnp.zeros((6, _L), jnp.float32)
    tab = tab.at[0:4, 0:10].set(object_emb[0:10, :].T)
    tab = tab.at[4:6, 0:10].set(state_emb[0:10, :].T)

    outv = _sc_embed(imgv, tab)

    y = outv.reshape(6, _H, _NWT, _NBT, 8, 128)
    y = y.transpose(0, 1, 2, 4, 3, 5)
    y = y.reshape(6, _H, _W, _B)
    return y.transpose(3, 0, 1, 2)
